# baseline (speedup 1.0000x reference)
import functools

import jax
import jax.numpy as jnp
from jax import lax
from jax.experimental import pallas as pl
from jax.experimental.pallas import tpu as pltpu
from jax.experimental.pallas import tpu_sc as plsc

_BATCH = 16384
_D = 16
_NC = 2
_NS = 16
_NW = _NC * _NS
_BPW = _BATCH // _NW
_NSLOT = 14


def _efm_body(user_hbm, item_hbm, ue_hbm, ie_hbm, uhe_hbm, ihe_hbm, out_hbm,
              idx_u_v, idx_i_v, ring, out_v, *sems):
    wid = lax.axis_index("s") * _NC + lax.axis_index("c")
    base = wid * _BPW

    pltpu.sync_copy(user_hbm.at[pl.ds(base, _BPW)], idx_u_v)
    pltpu.sync_copy(item_hbm.at[pl.ds(base, _BPW)], idx_i_v)

    tables = (ue_hbm, ie_hbm, uhe_hbm, ihe_hbm)
    lane = lax.iota(jnp.int32, 16)

    def fire_one(e, s):
        evec = jnp.zeros((16,), jnp.int32) + e
        ru = plsc.load_gather(idx_u_v, [evec])[0]
        ri = plsc.load_gather(idx_i_v, [evec])[0]
        for t, (tbl, r) in enumerate(
                zip(tables, (ru, ri, ru, ri))):
            c = pl.multiple_of((r >> 7) * 128, 128)
            pltpu.async_copy(
                tbl.at[:, pl.ds(c, 128)],
                ring.at[s, t],
                sems[s])

    def compute_one(e, s):
        evec = jnp.zeros((16,), jnp.int32) + e
        lu = plsc.load_gather(idx_u_v, [evec]) & 127
        li = plsc.load_gather(idx_i_v, [evec]) & 127
        svec_s = jnp.zeros((16,), jnp.int32) + s
        cols = []
        for t, l in zip(range(4), (lu, li, lu, li)):
            pvec = jnp.zeros((16,), jnp.int32) + t
            cols.append(plsc.load_gather(ring, [svec_s, pvec, lane, l]))
        u, i, uh, ih = cols
        ssum = jnp.sum(u * i + uh * ih)
        g = e >> 4
        r = e & 15
        sl = pl.ds(pl.multiple_of(g * 16, 16), 16)
        out_v[sl] = jnp.where(lane == r, ssum, out_v[sl])

    def super_body(S, carry):
        for s in range(_NSLOT):
            e = S * _NSLOT + s

            @pl.when((e >= _NSLOT) & (e < _BPW + _NSLOT))
            def _():
                for _ in range(len(tables)):
                    pltpu.make_async_copy(
                        ue_hbm.at[:, pl.ds(0, 128)],
                        ring.at[0, 0],
                        sems[s]).wait()
                compute_one(e - _NSLOT, s)

            @pl.when(e < _BPW)
            def _():
                fire_one(e, s)
        return carry

    lax.fori_loop(0, -(-(_BPW + _NSLOT) // _NSLOT), super_body, 0)

    pltpu.sync_copy(out_v, out_hbm.at[pl.ds(base, _BPW)])


@jax.jit
def kernel(user, item, user_emb, item_emb, user_h_emb, item_h_emb):
    mesh = plsc.VectorSubcoreMesh(core_axis_name="c", subcore_axis_name="s")
    run = pl.kernel(
        _efm_body,
        out_type=jax.ShapeDtypeStruct((_BATCH,), jnp.float32),
        mesh=mesh,
        scratch_types=[
            pltpu.VMEM((_BPW,), jnp.int32),
            pltpu.VMEM((_BPW,), jnp.int32),
            pltpu.VMEM((_NSLOT, 4, 16, 128), jnp.float32),
            pltpu.VMEM((_BPW,), jnp.float32),
        ] + [pltpu.SemaphoreType.DMA] * _NSLOT,
        compiler_params=pltpu.CompilerParams(
            needs_layout_passes=False, use_tc_tiling_on_sc=True),
    )
    return run(user, item, user_emb.T, item_emb.T, user_h_emb.T, item_h_emb.T)


# speedup vs baseline: 1.0127x; 1.0127x over previous
import jax
import jax.numpy as jnp
from jax import lax
from jax.experimental import pallas as pl
from jax.experimental.pallas import tpu as pltpu
from jax.experimental.pallas import tpu_sc as plsc

_BATCH = 16384
_D = 16
_NC = 2
_NS = 16
_NW = _NC * _NS
_BPW = _BATCH // _NW
_NSLOT = 12


def _efm_body(user_hbm, item_hbm, ue_hbm, ie_hbm, uhe_hbm, ihe_hbm, out_hbm,
              idx_u_v, idx_i_v, ring, out_v, *sems):
    wid = lax.axis_index("s") * _NC + lax.axis_index("c")
    base = wid * _BPW

    pltpu.sync_copy(user_hbm.at[pl.ds(base, _BPW)], idx_u_v)
    pltpu.sync_copy(item_hbm.at[pl.ds(base, _BPW)], idx_i_v)

    tables = (ue_hbm, ie_hbm, uhe_hbm, ihe_hbm)
    lane = lax.iota(jnp.int32, 16)

    def fire_one(e, s):
        evec = jnp.zeros((16,), jnp.int32) + e
        ru = plsc.load_gather(idx_u_v, [evec])[0]
        ri = plsc.load_gather(idx_i_v, [evec])[0]
        for t, (tbl, r) in enumerate(
                zip(tables, (ru, ri, ru, ri))):
            c = pl.multiple_of((r >> 7) * 128, 128)
            pltpu.async_copy(
                tbl.at[:, pl.ds(c, 128)],
                ring.at[s, t],
                sems[s])

    def compute_one(e, s):
        evec = jnp.zeros((16,), jnp.int32) + e
        lu = plsc.load_gather(idx_u_v, [evec]) & 127
        li = plsc.load_gather(idx_i_v, [evec]) & 127
        svec_s = jnp.zeros((16,), jnp.int32) + s
        cols = []
        for t, l in zip(range(4), (lu, li, lu, li)):
            pvec = jnp.zeros((16,), jnp.int32) + t
            cols.append(plsc.load_gather(ring, [svec_s, pvec, lane, l]))
        u, i, uh, ih = cols
        ssum = jnp.sum(u * i + uh * ih)
        g = e >> 4
        r = e & 15
        sl = pl.ds(pl.multiple_of(g * 16, 16), 16)
        out_v[sl] = jnp.where(lane == r, ssum, out_v[sl])

    def super_body(S, carry):
        for s in range(_NSLOT):
            e = S * _NSLOT + s

            @pl.when((e >= _NSLOT) & (e < _BPW + _NSLOT))
            def _():
                for _ in range(len(tables)):
                    pltpu.make_async_copy(
                        ue_hbm.at[:, pl.ds(0, 128)],
                        ring.at[0, 0],
                        sems[s]).wait()
                compute_one(e - _NSLOT, s)

            @pl.when(e < _BPW)
            def _():
                fire_one(e, s)
        return carry

    lax.fori_loop(0, -(-(_BPW + _NSLOT) // _NSLOT), super_body, 0)

    pltpu.sync_copy(out_v, out_hbm.at[pl.ds(base, _BPW)])


@jax.jit
def kernel(user, item, user_emb, item_emb, user_h_emb, item_h_emb):
    mesh = plsc.VectorSubcoreMesh(core_axis_name="c", subcore_axis_name="s")
    run = pl.kernel(
        _efm_body,
        out_type=jax.ShapeDtypeStruct((_BATCH,), jnp.float32),
        mesh=mesh,
        scratch_types=[
            pltpu.VMEM((_BPW,), jnp.int32),
            pltpu.VMEM((_BPW,), jnp.int32),
            pltpu.VMEM((_NSLOT, 4, 16, 128), jnp.float32),
            pltpu.VMEM((_BPW,), jnp.float32),
        ] + [pltpu.SemaphoreType.DMA] * _NSLOT,
        compiler_params=pltpu.CompilerParams(
            needs_layout_passes=False, use_tc_tiling_on_sc=True),
    )
    return run(user, item, user_emb.T, item_emb.T, user_h_emb.T, item_h_emb.T)
